# Initial kernel scaffold; baseline (speedup 1.0000x reference)
import functools

import jax
import jax.numpy as jnp
from jax import lax
from jax.experimental import pallas as pl
from jax.experimental.pallas import tpu as pltpu
from jax.experimental.pallas import tpu_sc as plsc

VOCAB = 100000
E = 100
B = 16384
C = 5
L = 16
NC, NS = 2, 16
NW = NC * NS
BPW = B // NW
CB = 64
ITERS = BPW // CB
PPI = CB * C
FIRE = 80
NFIRE = PPI // FIRE
NK = E // L
TAIL = E - NK * L


def _body(tgt_idx, ctx_idx, tgt_tab, ctx_tab, out,
          tgt_idx_v, ctx_idx_v, tgt_rows_v, ctx_rows_v, out_v,
          sem_t, sem_c):
    wid = lax.axis_index("s") * NC + lax.axis_index("c")
    iota = lax.iota(jnp.int32, L)
    tail_cols = jnp.minimum(NK * L + iota, E - 1)
    tail_keep = (iota < TAIL)
    lane15 = (iota == L - 1)

    pltpu.sync_copy(tgt_idx.at[pl.ds(wid * ITERS, ITERS), :], tgt_idx_v)
    pltpu.sync_copy(ctx_idx.at[pl.ds(wid * ITERS * NFIRE, ITERS * NFIRE), :],
                    ctx_idx_v)

    @pl.loop(0, ITERS)
    def _iter(it):
        d_t = pltpu.async_copy(tgt_tab.at[tgt_idx_v.at[it]], tgt_rows_v, sem_t)
        descs = [
            pltpu.async_copy(ctx_tab.at[ctx_idx_v.at[it * NFIRE + f]],
                             ctx_rows_v.at[pl.ds(f * FIRE, FIRE), :], sem_c)
            for f in range(NFIRE)
        ]
        d_t.wait()
        for d in descs:
            d.wait()

        @pl.loop(0, CB)
        def _row(b):
            b_vec = jnp.full((L,), b, dtype=jnp.int32)
            w = [tgt_rows_v[b, pl.ds(k * L, L)] for k in range(NK)]
            w_tail = plsc.load_gather(tgt_rows_v, [b_vec, tail_cols])
            w_tail = jnp.where(tail_keep, w_tail, 0.0)
            for c in range(C):
                p = b * C + c
                p_vec = jnp.full((L,), p, dtype=jnp.int32)
                acc = w[0] * ctx_rows_v[p, pl.ds(0, L)]
                for k in range(1, NK):
                    acc = acc + w[k] * ctx_rows_v[p, pl.ds(k * L, L)]
                x_tail = plsc.load_gather(ctx_rows_v, [p_vec, tail_cols])
                acc = acc + w_tail * x_tail
                s = plsc.cumsum(acc)
                gp = it * PPI + p
                plsc.store_scatter(out_v, [jnp.full((L,), gp, jnp.int32)],
                                   s, mask=lane15)

    pltpu.sync_copy(out_v, out.at[pl.ds(wid * BPW * C, BPW * C)])


def kernel(target, context, target_table, context_table):
    tgt_idx = target.reshape(B // CB, CB).astype(jnp.int32)
    ctx_idx = context.reshape(B * C // FIRE, FIRE).astype(jnp.int32)
    mesh = plsc.VectorSubcoreMesh(core_axis_name="c", subcore_axis_name="s",
                                  num_cores=NC, num_subcores=NS)
    run = pl.kernel(
        _body,
        out_type=jax.ShapeDtypeStruct((B * C,), jnp.float32),
        mesh=mesh,
        scratch_types=[
            pltpu.VMEM((ITERS, CB), jnp.int32),
            pltpu.VMEM((ITERS * NFIRE, FIRE), jnp.int32),
            pltpu.VMEM((CB, E), jnp.float32),
            pltpu.VMEM((PPI, E), jnp.float32),
            pltpu.VMEM((BPW * C,), jnp.float32),
            pltpu.SemaphoreType.DMA,
            pltpu.SemaphoreType.DMA,
        ],
    )
    out = run(tgt_idx, ctx_idx, target_table, context_table)
    return out.reshape(B, C)


# speedup vs baseline: 1.1819x; 1.1819x over previous
import functools

import jax
import jax.numpy as jnp
from jax import lax
from jax.experimental import pallas as pl
from jax.experimental.pallas import tpu as pltpu
from jax.experimental.pallas import tpu_sc as plsc

VOCAB = 100000
E = 100
B = 16384
C = 5
L = 16
NC, NS = 2, 16
NW = NC * NS
BPW = B // NW
CB = 64
ITERS = BPW // CB
PPI = CB * C
FIRE = 80
NFIRE = PPI // FIRE
NK = E // L
TAIL = E - NK * L


def _body(tgt_idx, ctx_idx, tgt_tab, ctx_tab, out,
          tgt_idx_v, ctx_idx_v, tgt_rows_v, ctx_rows_v, out_v,
          sem_t, sem_c):
    wid = lax.axis_index("s") * NC + lax.axis_index("c")
    iota = lax.iota(jnp.int32, L)
    tail_keep = (iota >= L - TAIL)
    lane15 = (iota == L - 1)

    pltpu.sync_copy(tgt_idx.at[pl.ds(wid * ITERS, ITERS), :], tgt_idx_v)
    pltpu.sync_copy(ctx_idx.at[pl.ds(wid * ITERS * NFIRE, ITERS * NFIRE), :],
                    ctx_idx_v)

    @pl.loop(0, ITERS)
    def _iter(it):
        d_t = pltpu.async_copy(tgt_tab.at[tgt_idx_v.at[it]], tgt_rows_v, sem_t)
        descs = [
            pltpu.async_copy(ctx_tab.at[ctx_idx_v.at[it * NFIRE + f]],
                             ctx_rows_v.at[pl.ds(f * FIRE, FIRE), :], sem_c)
            for f in range(NFIRE)
        ]
        d_t.wait()
        for d in descs:
            d.wait()

        @pl.loop(0, CB)
        def _row(b):
            w = [tgt_rows_v[b, pl.ds(k * L, L)] for k in range(NK)]
            w_tail = tgt_rows_v[b, pl.ds(E - L, L)]
            w_tail = jnp.where(tail_keep, w_tail, 0.0)
            for c in range(C):
                p = b * C + c
                acc = w[0] * ctx_rows_v[p, pl.ds(0, L)]
                for k in range(1, NK):
                    acc = acc + w[k] * ctx_rows_v[p, pl.ds(k * L, L)]
                acc = acc + w_tail * ctx_rows_v[p, pl.ds(E - L, L)]
                s = plsc.cumsum(acc)
                gp = jnp.full((L,), it * PPI + p, dtype=jnp.int32)
                plsc.store_scatter(out_v, [gp], s, mask=lane15)

    pltpu.sync_copy(out_v, out.at[pl.ds(wid * BPW * C, BPW * C)])


def kernel(target, context, target_table, context_table):
    tgt_idx = target.reshape(B // CB, CB).astype(jnp.int32)
    ctx_idx = context.reshape(B * C // FIRE, FIRE).astype(jnp.int32)
    mesh = plsc.VectorSubcoreMesh(core_axis_name="c", subcore_axis_name="s",
                                  num_cores=NC, num_subcores=NS)
    run = pl.kernel(
        _body,
        out_type=jax.ShapeDtypeStruct((B * C,), jnp.float32),
        mesh=mesh,
        compiler_params=pltpu.CompilerParams(needs_layout_passes=False,
                                             use_tc_tiling_on_sc=False),
        scratch_types=[
            pltpu.VMEM((ITERS, CB), jnp.int32),
            pltpu.VMEM((ITERS * NFIRE, FIRE), jnp.int32),
            pltpu.VMEM((CB, E), jnp.float32),
            pltpu.VMEM((PPI, E), jnp.float32),
            pltpu.VMEM((BPW * C,), jnp.float32),
            pltpu.SemaphoreType.DMA,
            pltpu.SemaphoreType.DMA,
        ],
    )
    out = run(tgt_idx, ctx_idx, target_table, context_table)
    return out.reshape(B, C)


# speedup vs baseline: 1.3350x; 1.1295x over previous
import jax
import jax.numpy as jnp
from jax import lax
from jax.experimental import pallas as pl
from jax.experimental.pallas import tpu as pltpu
from jax.experimental.pallas import tpu_sc as plsc

VOCAB = 100000
E = 100
EP = 128
B = 16384
C = 5
L = 16
NC, NS = 2, 16
NW = NC * NS
BPW = B // NW
CB = 128
ITERS = BPW // CB
PPI = CB * C
NFIRE = PPI // CB
NK = 7


def _body(tgt_idx, ctx_idx, tgt_tab, ctx_tab, out,
          tgt_idx_v, ctx_idx_v, tgt_rows_v, ctx_rows_v, out_v,
          sem_t, sem_c):
    wid = lax.axis_index("s") * NC + lax.axis_index("c")
    iota = lax.iota(jnp.int32, L)
    lane15 = (iota == L - 1)

    pltpu.sync_copy(tgt_idx.at[pl.ds(wid * BPW, BPW)], tgt_idx_v)
    pltpu.sync_copy(ctx_idx.at[pl.ds(wid * BPW * C, BPW * C)], ctx_idx_v)

    @pl.loop(0, ITERS)
    def _iter(it):
        d_t = pltpu.async_copy(
            tgt_tab.at[tgt_idx_v.at[pl.ds(it * CB, CB)]], tgt_rows_v, sem_t)
        descs = [
            pltpu.async_copy(
                ctx_tab.at[ctx_idx_v.at[pl.ds(it * PPI + f * CB, CB)]],
                ctx_rows_v.at[pl.ds(f * CB, CB), :], sem_c)
            for f in range(NFIRE)
        ]
        d_t.wait()
        for d in descs:
            d.wait()

        @pl.loop(0, CB)
        def _row(b):
            w = [tgt_rows_v[b, pl.ds(k * L, L)] for k in range(NK)]
            for c in range(C):
                p = b * C + c
                acc = w[0] * ctx_rows_v[p, pl.ds(0, L)]
                for k in range(1, NK):
                    acc = acc + w[k] * ctx_rows_v[p, pl.ds(k * L, L)]
                s = plsc.cumsum(acc)
                gp = jnp.full((L,), it * PPI + p, dtype=jnp.int32)
                plsc.store_scatter(out_v, [gp], s, mask=lane15)

    pltpu.sync_copy(out_v, out.at[pl.ds(wid * BPW * C, BPW * C)])


def kernel(target, context, target_table, context_table):
    tgt_idx = target.reshape(B).astype(jnp.int32)
    ctx_idx = context.reshape(B * C).astype(jnp.int32)
    tgt_tab = jnp.pad(target_table, ((0, 0), (0, EP - E)))
    ctx_tab = jnp.pad(context_table, ((0, 0), (0, EP - E)))
    mesh = plsc.VectorSubcoreMesh(core_axis_name="c", subcore_axis_name="s",
                                  num_cores=NC, num_subcores=NS)
    run = pl.kernel(
        _body,
        out_type=jax.ShapeDtypeStruct((B * C,), jnp.float32),
        mesh=mesh,
        compiler_params=pltpu.CompilerParams(needs_layout_passes=False),
        scratch_types=[
            pltpu.VMEM((BPW,), jnp.int32),
            pltpu.VMEM((BPW * C,), jnp.int32),
            pltpu.VMEM((CB, EP), jnp.float32),
            pltpu.VMEM((PPI, EP), jnp.float32),
            pltpu.VMEM((BPW * C,), jnp.float32),
            pltpu.SemaphoreType.DMA,
            pltpu.SemaphoreType.DMA,
        ],
    )
    out = run(tgt_idx, ctx_idx, tgt_tab, ctx_tab)
    return out.reshape(B, C)


# speedup vs baseline: 1.3352x; 1.0001x over previous
import jax
import jax.numpy as jnp
from jax import lax
from jax.experimental import pallas as pl
from jax.experimental.pallas import tpu as pltpu
from jax.experimental.pallas import tpu_sc as plsc

VOCAB = 100000
E = 100
EP = 128
B = 16384
C = 5
L = 16
NC, NS = 2, 16
NW = NC * NS
BPW = B // NW
CB = 128
ITERS = BPW // CB
PPI = CB * C
NFIRE = PPI // CB
NK = 7


def _body(tgt_idx, ctx_idx, tgt_tab, ctx_tab, out,
          tgt_idx_v, ctx_idx_v, tgt_rows_v, ctx_rows_v, out_v,
          sem_t, sem_c):
    wid = lax.axis_index("s") * NC + lax.axis_index("c")
    iota = lax.iota(jnp.int32, L)
    lane15 = (iota == L - 1)

    pltpu.sync_copy(tgt_idx.at[pl.ds(wid * BPW, BPW)], tgt_idx_v)
    pltpu.sync_copy(ctx_idx.at[pl.ds(wid * BPW * C, BPW * C)], ctx_idx_v)

    @pl.loop(0, ITERS)
    def _iter(it):
        d_t = pltpu.async_copy(
            tgt_tab.at[tgt_idx_v.at[pl.ds(it * CB, CB)]], tgt_rows_v, sem_t)
        descs = [
            pltpu.async_copy(
                ctx_tab.at[ctx_idx_v.at[pl.ds(it * PPI + f * CB, CB)]],
                ctx_rows_v.at[pl.ds(f * CB, CB), :], sem_c)
            for f in range(NFIRE)
        ]
        d_t.wait()
        for d in descs:
            d.wait()

        @pl.loop(0, CB)
        def _row(b):
            w = [tgt_rows_v[b, pl.ds(k * L, L)] for k in range(NK)]
            for c in range(C):
                p = b * C + c
                acc = w[0] * ctx_rows_v[p, pl.ds(0, L)]
                for k in range(1, NK):
                    acc = acc + w[k] * ctx_rows_v[p, pl.ds(k * L, L)]
                s = plsc.cumsum(acc)
                gp = jnp.full((L,), it * PPI + p, dtype=jnp.int32)
                plsc.store_scatter(out_v, [gp], s, mask=lane15)

    pltpu.sync_copy(out_v, out.at[pl.ds(wid * BPW * C, BPW * C)])


def kernel(target, context, target_table, context_table):
    tgt_idx = target.reshape(B).astype(jnp.int32)
    ctx_idx = context.reshape(B * C).astype(jnp.int32)
    tgt_tab = jnp.pad(target_table, ((0, 0), (0, EP - E)))
    ctx_tab = jnp.pad(context_table, ((0, 0), (0, EP - E)))
    mesh = plsc.VectorSubcoreMesh(core_axis_name="c", subcore_axis_name="s",
                                  num_cores=NC, num_subcores=NS)
    run = pl.kernel(
        _body,
        out_type=jax.ShapeDtypeStruct((B * C,), jnp.float32),
        mesh=mesh,
        compiler_params=pltpu.CompilerParams(needs_layout_passes=False,
                                             use_tc_tiling_on_sc=True),
        scratch_types=[
            pltpu.VMEM((BPW,), jnp.int32),
            pltpu.VMEM((BPW * C,), jnp.int32),
            pltpu.VMEM((CB, EP), jnp.float32),
            pltpu.VMEM((PPI, EP), jnp.float32),
            pltpu.VMEM((BPW * C,), jnp.float32),
            pltpu.SemaphoreType.DMA,
            pltpu.SemaphoreType.DMA,
        ],
    )
    out = run(tgt_idx, ctx_idx, tgt_tab, ctx_tab)
    return out.reshape(B, C)


# speedup vs baseline: 2.1225x; 1.5896x over previous
import jax
import jax.numpy as jnp
from jax import lax
from jax.experimental import pallas as pl
from jax.experimental.pallas import tpu as pltpu
from jax.experimental.pallas import tpu_sc as plsc

VOCAB = 100000
E = 100
EP = 128
B = 16384
C = 5
L = 16
NC, NS = 2, 16
NW = NC * NS
BPW = B // NW
CB = 128
ITERS = BPW // CB
PPI = CB * C
NFIRE = PPI // CB
NK = 7


def _body(tgt_idx, ctx_idx, tgt_tab, ctx_tab, out,
          tgt_idx_v, ctx_idx_v, tgt_rows_v, ctx_rows_v, out_v,
          sem_t, sem_c):
    wid = lax.axis_index("s") * NC + lax.axis_index("c")
    iota = lax.iota(jnp.int32, L)
    lane15 = (iota == L - 1)

    pltpu.sync_copy(tgt_idx.at[pl.ds(wid * BPW, BPW)], tgt_idx_v)
    pltpu.sync_copy(ctx_idx.at[pl.ds(wid * BPW * C, BPW * C)], ctx_idx_v)

    @pl.loop(0, ITERS)
    def _iter(it):
        d_t = pltpu.async_copy(
            tgt_tab.at[tgt_idx_v.at[pl.ds(it * CB, CB)]], tgt_rows_v, sem_t)
        descs = [
            pltpu.async_copy(
                ctx_tab.at[ctx_idx_v.at[pl.ds(it * PPI + f * CB, CB)]],
                ctx_rows_v.at[pl.ds(f * CB, CB), :], sem_c)
            for f in range(NFIRE)
        ]
        d_t.wait()
        for d in descs:
            d.wait()

        @pl.loop(0, CB)
        def _row(b):
            w = [tgt_rows_v[b, pl.ds(k * L, L)] for k in range(NK)]
            for c in range(C):
                p = b * C + c
                acc = w[0] * ctx_rows_v[p, pl.ds(0, L)]
                for k in range(1, NK):
                    acc = acc + w[k] * ctx_rows_v[p, pl.ds(k * L, L)]
                s = plsc.cumsum(acc)
                gp = jnp.full((L,), it * PPI + p, dtype=jnp.int32)
                plsc.store_scatter(out_v, [gp], s, mask=lane15)

    pltpu.sync_copy(out_v, out.at[pl.ds(wid * BPW * C, BPW * C)])


PAD_BS = 1000


def _pad_body(t_in, c_in, t_out, c_out):
    zeros = jnp.zeros((PAD_BS, EP - E), jnp.float32)
    t_out[...] = jnp.concatenate([t_in[...], zeros], axis=1)
    c_out[...] = jnp.concatenate([c_in[...], zeros], axis=1)


def _pad_tables(target_table, context_table):
    grid = (VOCAB // PAD_BS,)
    return pl.pallas_call(
        _pad_body,
        grid=grid,
        in_specs=[pl.BlockSpec((PAD_BS, E), lambda i: (i, 0))] * 2,
        out_specs=[pl.BlockSpec((PAD_BS, EP), lambda i: (i, 0))] * 2,
        out_shape=[jax.ShapeDtypeStruct((VOCAB, EP), jnp.float32)] * 2,
    )(target_table, context_table)


def kernel(target, context, target_table, context_table):
    tgt_idx = target.reshape(B).astype(jnp.int32)
    ctx_idx = context.reshape(B * C).astype(jnp.int32)
    tgt_tab, ctx_tab = _pad_tables(target_table, context_table)
    mesh = plsc.VectorSubcoreMesh(core_axis_name="c", subcore_axis_name="s",
                                  num_cores=NC, num_subcores=NS)
    run = pl.kernel(
        _body,
        out_type=jax.ShapeDtypeStruct((B * C,), jnp.float32),
        mesh=mesh,
        compiler_params=pltpu.CompilerParams(needs_layout_passes=False,
                                             use_tc_tiling_on_sc=True),
        scratch_types=[
            pltpu.VMEM((BPW,), jnp.int32),
            pltpu.VMEM((BPW * C,), jnp.int32),
            pltpu.VMEM((CB, EP), jnp.float32),
            pltpu.VMEM((PPI, EP), jnp.float32),
            pltpu.VMEM((BPW * C,), jnp.float32),
            pltpu.SemaphoreType.DMA,
            pltpu.SemaphoreType.DMA,
        ],
    )
    out = run(tgt_idx, ctx_idx, tgt_tab, ctx_tab)
    return out.reshape(B, C)


# speedup vs baseline: 2.1248x; 1.0011x over previous
import jax
import jax.numpy as jnp
from jax import lax
from jax.experimental import pallas as pl
from jax.experimental.pallas import tpu as pltpu
from jax.experimental.pallas import tpu_sc as plsc

VOCAB = 100000
E = 100
EP = 128
B = 16384
C = 5
L = 16
NC, NS = 2, 16
NW = NC * NS
BPW = B // NW
PW = BPW * C
CB = 64
ITERS = BPW // CB
PPI = CB * C
FIRE = 128
NCF = PW // FIRE
NTF = BPW // FIRE
RING = 4 * FIRE
NK = 7


def _body(tgt_idx, ctx_idx, tgt_tab, ctx_tab, out,
          tgt_idx_v, ctx_idx_v, tgt_rows_v, ctx_rows_v, out_v,
          sem_t, sem_c):
    wid = lax.axis_index("s") * NC + lax.axis_index("c")
    iota = lax.iota(jnp.int32, L)
    lane15 = (iota == L - 1)

    pltpu.sync_copy(tgt_idx.at[pl.ds(wid * BPW, BPW)], tgt_idx_v)
    pltpu.sync_copy(ctx_idx.at[pl.ds(wid * PW, PW)], ctx_idx_v)

    def fire_ctx(k):
        return pltpu.async_copy(
            ctx_tab.at[ctx_idx_v.at[pl.ds(k * FIRE, FIRE)]],
            ctx_rows_v.at[pl.ds((k % 4) * FIRE, FIRE), :], sem_c)

    def fire_tgt(j):
        return pltpu.async_copy(
            tgt_tab.at[tgt_idx_v.at[pl.ds(j * FIRE, FIRE)]],
            tgt_rows_v.at[j % 2], sem_t)

    ctx_descs, tgt_descs = [], []
    n_issued = n_waited = 0
    nt_issued = nt_waited = 0
    for it in range(ITERS):
        k_allowed = min((320 * it) // FIRE + 4, NCF)
        while n_issued < k_allowed:
            ctx_descs.append(fire_ctx(n_issued))
            n_issued += 1
        if (it == 0 or it % 2 == 1) and nt_issued < NTF:
            tgt_descs.append(fire_tgt(nt_issued))
            nt_issued += 1
        k_needed = (320 * it + PPI - 1) // FIRE + 1
        while n_waited < k_needed:
            ctx_descs[n_waited].wait()
            n_waited += 1
        while nt_waited < (it // 2) + 1:
            tgt_descs[nt_waited].wait()
            nt_waited += 1

        buf = (it // 2) % 2
        roff = (it % 2) * CB

        @pl.loop(0, CB)
        def _row(b):
            w = [tgt_rows_v[buf, roff + b, pl.ds(k * L, L)]
                 for k in range(NK)]
            for c in range(C):
                p = it * PPI + b * C + c
                rb = p & (RING - 1)
                acc = w[0] * ctx_rows_v[rb, pl.ds(0, L)]
                for k in range(1, NK):
                    acc = acc + w[k] * ctx_rows_v[rb, pl.ds(k * L, L)]
                s = plsc.cumsum(acc)
                gp = jnp.full((L,), p, dtype=jnp.int32)
                plsc.store_scatter(out_v, [gp], s, mask=lane15)

    pltpu.sync_copy(out_v, out.at[pl.ds(wid * PW, PW)])


PAD_BS = 1000


def _pad_body(t_in, c_in, t_out, c_out):
    zeros = jnp.zeros((PAD_BS, EP - E), jnp.float32)
    t_out[...] = jnp.concatenate([t_in[...], zeros], axis=1)
    c_out[...] = jnp.concatenate([c_in[...], zeros], axis=1)


def _pad_tables(target_table, context_table):
    grid = (VOCAB // PAD_BS,)
    return pl.pallas_call(
        _pad_body,
        grid=grid,
        in_specs=[pl.BlockSpec((PAD_BS, E), lambda i: (i, 0))] * 2,
        out_specs=[pl.BlockSpec((PAD_BS, EP), lambda i: (i, 0))] * 2,
        out_shape=[jax.ShapeDtypeStruct((VOCAB, EP), jnp.float32)] * 2,
    )(target_table, context_table)


def kernel(target, context, target_table, context_table):
    tgt_idx = target.reshape(B).astype(jnp.int32)
    ctx_idx = context.reshape(B * C).astype(jnp.int32)
    tgt_tab, ctx_tab = _pad_tables(target_table, context_table)
    mesh = plsc.VectorSubcoreMesh(core_axis_name="c", subcore_axis_name="s",
                                  num_cores=NC, num_subcores=NS)
    run = pl.kernel(
        _body,
        out_type=jax.ShapeDtypeStruct((B * C,), jnp.float32),
        mesh=mesh,
        compiler_params=pltpu.CompilerParams(needs_layout_passes=False,
                                             use_tc_tiling_on_sc=True),
        scratch_types=[
            pltpu.VMEM((BPW,), jnp.int32),
            pltpu.VMEM((PW,), jnp.int32),
            pltpu.VMEM((2, 2 * CB, EP), jnp.float32),
            pltpu.VMEM((RING, EP), jnp.float32),
            pltpu.VMEM((PW,), jnp.float32),
            pltpu.SemaphoreType.DMA,
            pltpu.SemaphoreType.DMA,
        ],
    )
    out = run(tgt_idx, ctx_idx, tgt_tab, ctx_tab)
    return out.reshape(B, C)


# speedup vs baseline: 2.3621x; 1.1117x over previous
import jax
import jax.numpy as jnp
from jax import lax
from jax.experimental import pallas as pl
from jax.experimental.pallas import tpu as pltpu
from jax.experimental.pallas import tpu_sc as plsc

VOCAB = 100000
E = 100
EP = 128
B = 16384
C = 5
L = 16
NC, NS = 2, 16
NW = NC * NS
BPW = B // NW
PW = BPW * C
CB = 64
ITERS = BPW // CB
PPI = CB * C
FIRE = 128
NCF = PW // FIRE
NTF = BPW // FIRE
RING = 4 * FIRE
NK = 7


def _body(tgt_idx, ctx_idx, tgt_tab, ctx_tab, out,
          tgt_idx_v, ctx_idx_v, tgt_rows_v, ctx_rows_v, out_v,
          sem_t, sem_c):
    wid = lax.axis_index("s") * NC + lax.axis_index("c")
    iota = lax.iota(jnp.int32, L)
    lane15 = (iota == L - 1)

    pltpu.sync_copy(tgt_idx.at[pl.ds(wid * BPW, BPW)], tgt_idx_v)
    pltpu.sync_copy(ctx_idx.at[pl.ds(wid * PW, PW)], ctx_idx_v)

    def fire_ctx(k):
        return pltpu.async_copy(
            ctx_tab.at[ctx_idx_v.at[pl.ds(k * FIRE, FIRE)]],
            ctx_rows_v.at[pl.ds((k % 4) * FIRE, FIRE), :], sem_c)

    def fire_tgt(j):
        return pltpu.async_copy(
            tgt_tab.at[tgt_idx_v.at[pl.ds(j * FIRE, FIRE)]],
            tgt_rows_v.at[j % 2], sem_t)

    ctx_descs, tgt_descs = [], []
    n_issued = n_waited = 0
    nt_issued = nt_waited = 0
    for it in range(ITERS):
        k_allowed = min((320 * it) // FIRE + 4, NCF)
        while n_issued < k_allowed:
            ctx_descs.append(fire_ctx(n_issued))
            n_issued += 1
        if (it == 0 or it % 2 == 1) and nt_issued < NTF:
            tgt_descs.append(fire_tgt(nt_issued))
            nt_issued += 1
        k_needed = (320 * it + PPI - 1) // FIRE + 1
        while n_waited < k_needed:
            ctx_descs[n_waited].wait()
            n_waited += 1
        while nt_waited < (it // 2) + 1:
            tgt_descs[nt_waited].wait()
            nt_waited += 1

        buf = (it // 2) % 2
        roff = (it % 2) * CB

        @plsc.parallel_loop(0, CB, unroll=2)
        def _row(b):
            w = [tgt_rows_v[buf, roff + b, pl.ds(k * L, L)]
                 for k in range(NK)]
            for c in range(C):
                p = it * PPI + b * C + c
                rb = p & (RING - 1)
                prod = [w[k] * ctx_rows_v[rb, pl.ds(k * L, L)]
                        for k in range(NK)]
                while len(prod) > 1:
                    prod = [prod[i] + prod[i + 1]
                            for i in range(0, len(prod) - 1, 2)] + (
                        [prod[-1]] if len(prod) % 2 else [])
                s = plsc.cumsum(prod[0])
                gp = jnp.full((L,), p, dtype=jnp.int32)
                plsc.store_scatter(out_v, [gp], s, mask=lane15)

    pltpu.sync_copy(out_v, out.at[pl.ds(wid * PW, PW)])


PAD_BS = 1000


def _pad_body(t_in, c_in, t_out, c_out):
    zeros = jnp.zeros((PAD_BS, EP - E), jnp.float32)
    t_out[...] = jnp.concatenate([t_in[...], zeros], axis=1)
    c_out[...] = jnp.concatenate([c_in[...], zeros], axis=1)


def _pad_tables(target_table, context_table):
    grid = (VOCAB // PAD_BS,)
    return pl.pallas_call(
        _pad_body,
        grid=grid,
        in_specs=[pl.BlockSpec((PAD_BS, E), lambda i: (i, 0))] * 2,
        out_specs=[pl.BlockSpec((PAD_BS, EP), lambda i: (i, 0))] * 2,
        out_shape=[jax.ShapeDtypeStruct((VOCAB, EP), jnp.float32)] * 2,
    )(target_table, context_table)


def kernel(target, context, target_table, context_table):
    tgt_idx = target.reshape(B).astype(jnp.int32)
    ctx_idx = context.reshape(B * C).astype(jnp.int32)
    tgt_tab, ctx_tab = _pad_tables(target_table, context_table)
    mesh = plsc.VectorSubcoreMesh(core_axis_name="c", subcore_axis_name="s",
                                  num_cores=NC, num_subcores=NS)
    run = pl.kernel(
        _body,
        out_type=jax.ShapeDtypeStruct((B * C,), jnp.float32),
        mesh=mesh,
        compiler_params=pltpu.CompilerParams(needs_layout_passes=False,
                                             use_tc_tiling_on_sc=True),
        scratch_types=[
            pltpu.VMEM((BPW,), jnp.int32),
            pltpu.VMEM((PW,), jnp.int32),
            pltpu.VMEM((2, 2 * CB, EP), jnp.float32),
            pltpu.VMEM((RING, EP), jnp.float32),
            pltpu.VMEM((PW,), jnp.float32),
            pltpu.SemaphoreType.DMA,
            pltpu.SemaphoreType.DMA,
        ],
    )
    out = run(tgt_idx, ctx_idx, tgt_tab, ctx_tab)
    return out.reshape(B, C)


# speedup vs baseline: 3.9433x; 1.6694x over previous
import jax
import jax.numpy as jnp
from jax import lax
from jax.experimental import pallas as pl
from jax.experimental.pallas import tpu as pltpu
from jax.experimental.pallas import tpu_sc as plsc

VOCAB = 100000
E = 100
B = 16384
C = 5
L = 16
NC, NS = 2, 16
NW = NC * NS
BPW = B // NW
PW = BPW * C
CB = 64
ITERS = BPW // CB
PPI = CB * C
NK = 6
TAIL = E - NK * L


def _body(tgt_idx, ctx_idx, tgt_tab, ctx_tab, out,
          tgt_idx_v, ctx_idx_v, tgt_rows_v, ctx_rows_v, out_v,
          sem_t0, sem_t1, sem_c0, sem_c1):
    wid = lax.axis_index("s") * NC + lax.axis_index("c")
    iota = lax.iota(jnp.int32, L)
    lane15 = (iota == L - 1)
    tail_keep = (iota >= L - TAIL)

    pltpu.sync_copy(tgt_idx.at[pl.ds(wid * BPW, BPW)], tgt_idx_v)
    pltpu.sync_copy(ctx_idx.at[pl.ds(wid * PW, PW)], ctx_idx_v)

    def fire_ctx(it1, buf, sem):
        @plsc.parallel_loop(0, PPI // L)
        def _issue(g):
            vec = ctx_idx_v[pl.ds(it1 * PPI + g * L, L)]
            for j in range(L):
                pltpu.async_copy(ctx_tab.at[pl.ds(vec[j], 1), :],
                                 ctx_rows_v.at[buf, pl.ds(g * L + j, 1), :],
                                 sem)

    def fire_tgt(it1, buf, sem):
        @plsc.parallel_loop(0, CB // L)
        def _issue(g):
            vec = tgt_idx_v[pl.ds(it1 * CB + g * L, L)]
            for j in range(L):
                pltpu.async_copy(tgt_tab.at[pl.ds(vec[j], 1), :],
                                 tgt_rows_v.at[buf, pl.ds(g * L + j, 1), :],
                                 sem)

    def drain(buf, sem_t, sem_c):
        pltpu.make_async_copy(tgt_tab.at[pl.ds(0, CB), :],
                              tgt_rows_v.at[buf], sem_t).wait()
        pltpu.make_async_copy(ctx_tab.at[pl.ds(0, PPI), :],
                              ctx_rows_v.at[buf], sem_c).wait()

    def compute(it, buf):
        @plsc.parallel_loop(0, CB, unroll=2)
        def _row(b):
            w = [tgt_rows_v[buf, b, pl.ds(k * L, L)] for k in range(NK)]
            w_tail = tgt_rows_v[buf, b, pl.ds(E - L, L)]
            for c in range(C):
                q = b * C + c
                prod = [w[k] * ctx_rows_v[buf, q, pl.ds(k * L, L)]
                        for k in range(NK)]
                prod.append(jnp.where(
                    tail_keep,
                    w_tail * ctx_rows_v[buf, q, pl.ds(E - L, L)], 0.0))
                while len(prod) > 1:
                    prod = [prod[i] + prod[i + 1]
                            for i in range(0, len(prod) - 1, 2)] + (
                        [prod[-1]] if len(prod) % 2 else [])
                s = plsc.cumsum(prod[0])
                gp = jnp.full((L,), it * PPI + q, dtype=jnp.int32)
                plsc.store_scatter(out_v, [gp], s, mask=lane15)

    fire_tgt(0, 0, sem_t0)
    fire_ctx(0, 0, sem_c0)

    @pl.loop(0, ITERS, step=2)
    def _it2(it):
        fire_tgt(it + 1, 1, sem_t1)
        fire_ctx(it + 1, 1, sem_c1)
        drain(0, sem_t0, sem_c0)
        compute(it, 0)

        @pl.when(it + 2 < ITERS)
        def _():
            fire_tgt(it + 2, 0, sem_t0)
            fire_ctx(it + 2, 0, sem_c0)

        drain(1, sem_t1, sem_c1)
        compute(it + 1, 1)

    pltpu.sync_copy(out_v, out.at[pl.ds(wid * PW, PW)])


def kernel(target, context, target_table, context_table):
    tgt_idx = target.reshape(B).astype(jnp.int32)
    ctx_idx = context.reshape(B * C).astype(jnp.int32)
    mesh = plsc.VectorSubcoreMesh(core_axis_name="c", subcore_axis_name="s",
                                  num_cores=NC, num_subcores=NS)
    run = pl.kernel(
        _body,
        out_type=jax.ShapeDtypeStruct((B * C,), jnp.float32),
        mesh=mesh,
        compiler_params=pltpu.CompilerParams(needs_layout_passes=False,
                                             use_tc_tiling_on_sc=True),
        scratch_types=[
            pltpu.VMEM((BPW,), jnp.int32),
            pltpu.VMEM((PW,), jnp.int32),
            pltpu.VMEM((2, CB, E), jnp.float32),
            pltpu.VMEM((2, PPI, E), jnp.float32),
            pltpu.VMEM((PW,), jnp.float32),
            pltpu.SemaphoreType.DMA,
            pltpu.SemaphoreType.DMA,
            pltpu.SemaphoreType.DMA,
            pltpu.SemaphoreType.DMA,
        ],
    )
    out = run(tgt_idx, ctx_idx, target_table, context_table)
    return out.reshape(B, C)


# speedup vs baseline: 3.9483x; 1.0013x over previous
import jax
import jax.numpy as jnp
from jax import lax
from jax.experimental import pallas as pl
from jax.experimental.pallas import tpu as pltpu
from jax.experimental.pallas import tpu_sc as plsc

VOCAB = 100000
E = 100
B = 16384
C = 5
L = 16
NC, NS = 2, 16
NW = NC * NS
BPW = B // NW
PW = BPW * C
CB = 64
ITERS = BPW // CB
PPI = CB * C
NK = 6
TAIL = E - NK * L


def _body(tgt_idx, ctx_idx, tgt_tab, ctx_tab, out,
          tgt_idx_v, ctx_idx_v, tgt_rows_v, ctx_rows_v, out_v,
          sem_t0, sem_t1, sem_c0, sem_c1):
    wid = lax.axis_index("s") * NC + lax.axis_index("c")
    iota = lax.iota(jnp.int32, L)
    lane15 = (iota == L - 1)
    tail_keep = (iota >= L - TAIL)

    pltpu.sync_copy(tgt_idx.at[pl.ds(wid * BPW, BPW)], tgt_idx_v)
    pltpu.sync_copy(ctx_idx.at[pl.ds(wid * PW, PW)], ctx_idx_v)

    def fire_ctx(it1, buf, sem):
        @plsc.parallel_loop(0, PPI // L)
        def _issue(g):
            vec = ctx_idx_v[pl.ds(it1 * PPI + g * L, L)]
            for j in range(L):
                pltpu.async_copy(ctx_tab.at[pl.ds(vec[j], 1), :],
                                 ctx_rows_v.at[buf, pl.ds(g * L + j, 1), :],
                                 sem)

    def fire_tgt(it1, buf, sem):
        @plsc.parallel_loop(0, CB // L)
        def _issue(g):
            vec = tgt_idx_v[pl.ds(it1 * CB + g * L, L)]
            for j in range(L):
                pltpu.async_copy(tgt_tab.at[pl.ds(vec[j], 1), :],
                                 tgt_rows_v.at[buf, pl.ds(g * L + j, 1), :],
                                 sem)

    def drain(buf, sem_t, sem_c):
        pltpu.make_async_copy(tgt_tab.at[pl.ds(0, CB), :],
                              tgt_rows_v.at[buf], sem_t).wait()
        pltpu.make_async_copy(ctx_tab.at[pl.ds(0, PPI), :],
                              ctx_rows_v.at[buf], sem_c).wait()

    def compute(it, buf):
        @plsc.parallel_loop(0, CB, unroll=2)
        def _row(b):
            w = [tgt_rows_v[buf, b, pl.ds(k * L, L)] for k in range(NK)]
            w_tail = tgt_rows_v[buf, b, pl.ds(E - L, L)]
            for c in range(C):
                q = b * C + c
                prod = [w[k] * ctx_rows_v[buf, q, pl.ds(k * L, L)]
                        for k in range(NK)]
                prod.append(jnp.where(
                    tail_keep,
                    w_tail * ctx_rows_v[buf, q, pl.ds(E - L, L)], 0.0))
                while len(prod) > 1:
                    prod = [prod[i] + prod[i + 1]
                            for i in range(0, len(prod) - 1, 2)] + (
                        [prod[-1]] if len(prod) % 2 else [])
                s = plsc.cumsum(prod[0])
                gp = jnp.full((L,), it * PPI + q, dtype=jnp.int32)
                plsc.store_scatter(out_v, [gp], s, mask=lane15)

    fire_tgt(0, 0, sem_t0)
    fire_ctx(0, 0, sem_c0)

    @pl.loop(0, ITERS, step=2)
    def _it2(it):
        fire_tgt(it + 1, 1, sem_t1)
        fire_ctx(it + 1, 1, sem_c1)
        drain(0, sem_t0, sem_c0)
        compute(it, 0)

        @pl.when(it + 2 < ITERS)
        def _():
            fire_tgt(it + 2, 0, sem_t0)
            fire_ctx(it + 2, 0, sem_c0)

        drain(1, sem_t1, sem_c1)
        compute(it + 1, 1)

    pltpu.sync_copy(out_v, out.at[pl.ds(wid * PW, PW)])


def kernel(target, context, target_table, context_table):
    tgt_idx = target.reshape(B).astype(jnp.int32)
    ctx_idx = context.reshape(B * C).astype(jnp.int32)
    mesh = plsc.VectorSubcoreMesh(core_axis_name="c", subcore_axis_name="s",
                                  num_cores=NC, num_subcores=NS)
    run = pl.kernel(
        _body,
        out_type=jax.ShapeDtypeStruct((B * C,), jnp.float32),
        mesh=mesh,
        compiler_params=pltpu.CompilerParams(needs_layout_passes=False,
                                             use_tc_tiling_on_sc=True,
                                             disable_bounds_checks=True,
                                             disable_semaphore_checks=True),
        scratch_types=[
            pltpu.VMEM((BPW,), jnp.int32),
            pltpu.VMEM((PW,), jnp.int32),
            pltpu.VMEM((2, CB, E), jnp.float32),
            pltpu.VMEM((2, PPI, E), jnp.float32),
            pltpu.VMEM((PW,), jnp.float32),
            pltpu.SemaphoreType.DMA,
            pltpu.SemaphoreType.DMA,
            pltpu.SemaphoreType.DMA,
            pltpu.SemaphoreType.DMA,
        ],
    )
    out = run(tgt_idx, ctx_idx, target_table, context_table)
    return out.reshape(B, C)


# speedup vs baseline: 3.9492x; 1.0002x over previous
import jax
import jax.numpy as jnp
from jax import lax
from jax.experimental import pallas as pl
from jax.experimental.pallas import tpu as pltpu
from jax.experimental.pallas import tpu_sc as plsc

VOCAB = 100000
E = 100
B = 16384
C = 5
L = 16
NC, NS = 2, 16
NW = NC * NS
BPW = B // NW
PW = BPW * C
CB = 64
ITERS = BPW // CB
PPI = CB * C
NK = 6
TAIL = E - NK * L


def _body(tgt_idx, ctx_idx, tgt_tab, ctx_tab, out,
          tgt_idx_v, ctx_idx_v, tgt_rows_v, ctx_rows_v, out_v,
          sem_t0, sem_t1, sem_c0, sem_c1):
    wid = lax.axis_index("s") * NC + lax.axis_index("c")
    iota = lax.iota(jnp.int32, L)
    lane15 = (iota == L - 1)
    tail_keep = (iota >= L - TAIL)

    pltpu.sync_copy(tgt_idx.at[pl.ds(wid * BPW, BPW)], tgt_idx_v)
    pltpu.sync_copy(ctx_idx.at[pl.ds(wid * PW, PW)], ctx_idx_v)

    def fire_ctx(it1, buf, sem):
        @plsc.parallel_loop(0, PPI // L)
        def _issue(g):
            vec = ctx_idx_v[pl.ds(it1 * PPI + g * L, L)]
            for j in range(L):
                pltpu.async_copy(ctx_tab.at[pl.ds(vec[j], 1), :],
                                 ctx_rows_v.at[buf, pl.ds(g * L + j, 1), :],
                                 sem)

    def fire_tgt(it1, buf, sem):
        @plsc.parallel_loop(0, CB // L)
        def _issue(g):
            vec = tgt_idx_v[pl.ds(it1 * CB + g * L, L)]
            for j in range(L):
                pltpu.async_copy(tgt_tab.at[pl.ds(vec[j], 1), :],
                                 tgt_rows_v.at[buf, pl.ds(g * L + j, 1), :],
                                 sem)

    def drain(buf, sem_t, sem_c):
        pltpu.make_async_copy(tgt_tab.at[pl.ds(0, CB), :],
                              tgt_rows_v.at[buf], sem_t).wait()
        pltpu.make_async_copy(ctx_tab.at[pl.ds(0, PPI), :],
                              ctx_rows_v.at[buf], sem_c).wait()

    def compute(it, buf):
        @plsc.parallel_loop(0, CB, unroll=2)
        def _row(b):
            w = [tgt_rows_v[buf, b, pl.ds(k * L, L)] for k in range(NK)]
            w_tail = tgt_rows_v[buf, b, pl.ds(E - L, L)]
            for c in range(C):
                q = b * C + c
                prod = [w[k] * ctx_rows_v[buf, q, pl.ds(k * L, L)]
                        for k in range(NK)]
                prod.append(jnp.where(
                    tail_keep,
                    w_tail * ctx_rows_v[buf, q, pl.ds(E - L, L)], 0.0))
                while len(prod) > 1:
                    prod = [prod[i] + prod[i + 1]
                            for i in range(0, len(prod) - 1, 2)] + (
                        [prod[-1]] if len(prod) % 2 else [])
                s = plsc.cumsum(prod[0])
                gp = jnp.full((L,), it * PPI + q, dtype=jnp.int32)
                plsc.store_scatter(out_v, [gp], s, mask=lane15)

    fire_tgt(0, 0, sem_t0)
    fire_ctx(0, 0, sem_c0)

    @pl.loop(0, ITERS, step=2)
    def _it2(it):
        fire_tgt(it + 1, 1, sem_t1)
        fire_ctx(it + 1, 1, sem_c1)
        drain(0, sem_t0, sem_c0)
        compute(it, 0)

        @pl.when(it + 2 < ITERS)
        def _():
            fire_tgt(it + 2, 0, sem_t0)
            fire_ctx(it + 2, 0, sem_c0)

        drain(1, sem_t1, sem_c1)
        compute(it + 1, 1)

    pltpu.sync_copy(out_v, out.at[pl.ds(wid * PW, PW)])


def kernel(target, context, target_table, context_table):
    tgt_idx = target.reshape(B).astype(jnp.int32)
    ctx_idx = context.reshape(B * C).astype(jnp.int32)
    mesh = plsc.VectorSubcoreMesh(core_axis_name="c", subcore_axis_name="s",
                                  num_cores=NC, num_subcores=NS)
    run = pl.kernel(
        _body,
        out_type=jax.ShapeDtypeStruct((B * C,), jnp.float32),
        mesh=mesh,
        compiler_params=pltpu.CompilerParams(needs_layout_passes=False,
                                             use_tc_tiling_on_sc=True,
                                             disable_bounds_checks=True,
                                             disable_semaphore_checks=True,
                                             skip_device_barrier=True),
        scratch_types=[
            pltpu.VMEM((BPW,), jnp.int32),
            pltpu.VMEM((PW,), jnp.int32),
            pltpu.VMEM((2, CB, E), jnp.float32),
            pltpu.VMEM((2, PPI, E), jnp.float32),
            pltpu.VMEM((PW,), jnp.float32),
            pltpu.SemaphoreType.DMA,
            pltpu.SemaphoreType.DMA,
            pltpu.SemaphoreType.DMA,
            pltpu.SemaphoreType.DMA,
        ],
    )
    out = run(tgt_idx, ctx_idx, target_table, context_table)
    return out.reshape(B, C)
